# baseline (speedup 1.0000x reference)
import functools

import jax
import jax.numpy as jnp
from jax import lax
from jax.experimental import pallas as pl
from jax.experimental.pallas import tpu as pltpu
from jax.experimental.pallas import tpu_sc as plsc

N_BINS = 15
N = 1048576
L = 16
NC, NS = 2, 16
NW = NC * NS
CHUNK = N // NW
BLK = 8192
NB = CHUNK // BLK
VECS = BLK // L
ACC = 768
QSTRIDE = 256

_mesh = plsc.VectorSubcoreMesh(
    core_axis_name="c", subcore_axis_name="s", num_cores=NC, num_subcores=NS
)
_params = pltpu.CompilerParams(needs_layout_passes=False)


@functools.partial(
    pl.kernel,
    out_type=jax.ShapeDtypeStruct((NW * ACC,), jnp.float32),
    mesh=_mesh,
    compiler_params=_params,
    scratch_types=[
        pltpu.VMEM((BLK,), jnp.int32),
        pltpu.VMEM((BLK,), jnp.float32),
        pltpu.VMEM((BLK,), jnp.int32),
        pltpu.VMEM((L,), jnp.float32),
        pltpu.VMEM((ACC,), jnp.float32),
    ],
)
def _hist_kernel(yhs_hbm, phs_hbm, ys_hbm, bounds_hbm, out_hbm,
                 yv, pv, vv, btab, acc):
    wid = lax.axis_index("c") * NS + lax.axis_index("s")
    pltpu.sync_copy(bounds_hbm, btab)

    zeros = jnp.zeros((L,), jnp.float32)
    for v in range(ACC // L):
        acc[pl.ds(v * L, L)] = zeros

    lane = lax.iota(jnp.int32, L)
    ones = jnp.ones((L,), jnp.float32)

    def do_vec(i, carry):
        off = i * L
        p = pv[pl.ds(off, L)]
        yh = yv[pl.ds(off, L)]
        yy = vv[pl.ds(off, L)]
        t = p * jnp.float32(N_BINS)
        j0 = jnp.clip(t.astype(jnp.int32), 0, N_BINS - 1)
        lo = plsc.load_gather(btab, [j0])
        hi = plsc.load_gather(btab, [j0 + 1])
        j = j0 - (p <= lo).astype(jnp.int32) + (p > hi).astype(jnp.int32)
        j = jnp.clip(j, 0, N_BINS - 1)
        valid = p > jnp.float32(0.0)
        idx = j * L + lane
        correct = jnp.where(yh == yy, jnp.float32(1.0), jnp.float32(0.0))
        plsc.addupdate_scatter(acc, [idx], ones, mask=valid)
        plsc.addupdate_scatter(acc, [idx + QSTRIDE], correct, mask=valid)
        plsc.addupdate_scatter(acc, [idx + 2 * QSTRIDE], p, mask=valid)
        return carry

    for b in range(NB):
        base = wid * CHUNK + b * BLK
        pltpu.sync_copy(yhs_hbm.at[pl.ds(base, BLK)], yv)
        pltpu.sync_copy(phs_hbm.at[pl.ds(base, BLK)], pv)
        pltpu.sync_copy(ys_hbm.at[pl.ds(base, BLK)], vv)
        lax.fori_loop(0, VECS, do_vec, 0)

    pltpu.sync_copy(acc, out_hbm.at[pl.ds(wid * ACC, ACC)])


@functools.partial(
    pl.kernel,
    out_type=jax.ShapeDtypeStruct((L,), jnp.float32),
    mesh=_mesh,
    compiler_params=_params,
    scratch_types=[
        pltpu.VMEM((NW * ACC,), jnp.float32),
        pltpu.VMEM((ACC,), jnp.float32),
        pltpu.VMEM((L,), jnp.float32),
    ],
)
def _ece_kernel(parts_hbm, out_hbm, pv, acc, outv):
    wid = lax.axis_index("c") * NS + lax.axis_index("s")

    @pl.when(wid == 0)
    def _():
        pltpu.sync_copy(parts_hbm, pv)
        zeros = jnp.zeros((L,), jnp.float32)
        for v in range(ACC // L):
            acc[pl.ds(v * L, L)] = zeros

        def add_worker(w, carry):
            def add_vec(v, c2):
                o = v * L
                acc[pl.ds(o, L)] += pv[pl.ds(w * ACC + o, L)]
                return c2
            return lax.fori_loop(0, ACC // L, add_vec, carry)

        lax.fori_loop(0, NW, add_worker, 0)

        lane = lax.iota(jnp.int32, L)
        zeros = jnp.zeros((L,), jnp.float32)
        counts_v = zeros
        acc_v = zeros
        conf_v = zeros
        for j in range(N_BINS):
            sel = lane == j
            c = jnp.sum(acc[pl.ds(j * L, L)])
            a = jnp.sum(acc[pl.ds(QSTRIDE + j * L, L)])
            f = jnp.sum(acc[pl.ds(2 * QSTRIDE + j * L, L)])
            counts_v = jnp.where(sel, jnp.broadcast_to(c, (L,)), counts_v)
            acc_v = jnp.where(sel, jnp.broadcast_to(a, (L,)), acc_v)
            conf_v = jnp.where(sel, jnp.broadcast_to(f, (L,)), conf_v)
        ones = jnp.ones((L,), jnp.float32)
        ind = counts_v > jnp.float32(0.0)
        safe = jnp.where(ind, counts_v, ones)
        mean_acc = jnp.where(ind, acc_v / safe, acc_v)
        mean_conf = jnp.where(ind, conf_v / safe, conf_v)
        num = jnp.sum(counts_v * jnp.abs(mean_acc - mean_conf))
        tot = jnp.sum(counts_v)
        outv[...] = jnp.broadcast_to(num, (L,)) / jnp.broadcast_to(tot, (L,))
        pltpu.sync_copy(outv, out_hbm)


@jax.jit
def kernel(yhs, phs, ys):
    bounds = jnp.linspace(0.0, 1.0, N_BINS + 1).astype(jnp.float32)
    parts = _hist_kernel(yhs, phs, ys, bounds)
    ece_vec = _ece_kernel(parts)
    return ece_vec[0]


# speedup vs baseline: 1.0877x; 1.0877x over previous
import functools

import jax
import jax.numpy as jnp
from jax import lax
from jax.experimental import pallas as pl
from jax.experimental.pallas import tpu as pltpu
from jax.experimental.pallas import tpu_sc as plsc

N_BINS = 15
N = 1048576
L = 16
NC, NS = 2, 16
NW = NC * NS
CHUNK = N // NW
BLK = 8192
NB = CHUNK // BLK
VECS = BLK // L
ACC = 768
QSTRIDE = 256

_mesh = plsc.VectorSubcoreMesh(
    core_axis_name="c", subcore_axis_name="s", num_cores=NC, num_subcores=NS
)
_params = pltpu.CompilerParams(needs_layout_passes=False)


@functools.partial(
    pl.kernel,
    out_type=jax.ShapeDtypeStruct((NW * ACC,), jnp.float32),
    mesh=_mesh,
    compiler_params=_params,
    scratch_types=[
        pltpu.VMEM((2, BLK), jnp.int32),
        pltpu.VMEM((2, BLK), jnp.float32),
        pltpu.VMEM((2, BLK), jnp.int32),
        pltpu.VMEM((L,), jnp.float32),
        pltpu.VMEM((ACC,), jnp.float32),
        pltpu.SemaphoreType.DMA,
        pltpu.SemaphoreType.DMA,
    ],
)
def _hist_kernel(yhs_hbm, phs_hbm, ys_hbm, bounds_hbm, out_hbm,
                 yv, pv, vv, btab, acc, sem0, sem1):
    wid = lax.axis_index("c") * NS + lax.axis_index("s")
    pltpu.sync_copy(bounds_hbm, btab)

    zeros = jnp.zeros((L,), jnp.float32)
    for v in range(ACC // L):
        acc[pl.ds(v * L, L)] = zeros

    lane = lax.iota(jnp.int32, L)
    ones = jnp.ones((L,), jnp.float32)
    sems = [sem0, sem1]

    def start_block(b):
        base = wid * CHUNK + b * BLK
        s = b % 2
        sem = sems[s]
        return [
            pltpu.async_copy(yhs_hbm.at[pl.ds(base, BLK)], yv.at[s], sem),
            pltpu.async_copy(phs_hbm.at[pl.ds(base, BLK)], pv.at[s], sem),
            pltpu.async_copy(ys_hbm.at[pl.ds(base, BLK)], vv.at[s], sem),
        ]

    def do_vec(i, s):
        off = i * L
        p = pv[s, pl.ds(off, L)]
        yh = yv[s, pl.ds(off, L)]
        yy = vv[s, pl.ds(off, L)]
        t = p * jnp.float32(N_BINS)
        j0 = jnp.clip(t.astype(jnp.int32), 0, N_BINS - 1)
        lo = plsc.load_gather(btab, [j0])
        hi = plsc.load_gather(btab, [j0 + 1])
        j = j0 - (p <= lo).astype(jnp.int32) + (p > hi).astype(jnp.int32)
        j = jnp.clip(j, 0, N_BINS - 1)
        valid = p > jnp.float32(0.0)
        idx = j * L + lane
        correct = jnp.where(yh == yy, jnp.float32(1.0), jnp.float32(0.0))
        plsc.addupdate_scatter(acc, [idx], ones, mask=valid)
        plsc.addupdate_scatter(acc, [idx + QSTRIDE], correct, mask=valid)
        plsc.addupdate_scatter(acc, [idx + 2 * QSTRIDE], p, mask=valid)

    pending = start_block(0)
    for b in range(NB):
        for h in pending:
            h.wait()
        pending = start_block(b + 1) if b + 1 < NB else []
        s = b % 2

        def body(i, carry):
            do_vec(i, s)
            return carry

        lax.fori_loop(0, VECS, body, 0, unroll=8)

    pltpu.sync_copy(acc, out_hbm.at[pl.ds(wid * ACC, ACC)])


@functools.partial(
    pl.kernel,
    out_type=jax.ShapeDtypeStruct((L,), jnp.float32),
    mesh=_mesh,
    compiler_params=_params,
    scratch_types=[
        pltpu.VMEM((NW * ACC,), jnp.float32),
        pltpu.VMEM((ACC,), jnp.float32),
        pltpu.VMEM((L,), jnp.float32),
    ],
)
def _ece_kernel(parts_hbm, out_hbm, pv, acc, outv):
    wid = lax.axis_index("c") * NS + lax.axis_index("s")

    @pl.when(wid == 0)
    def _():
        pltpu.sync_copy(parts_hbm, pv)
        zeros = jnp.zeros((L,), jnp.float32)
        for v in range(ACC // L):
            acc[pl.ds(v * L, L)] = zeros

        def add_worker(w, carry):
            for v in range(ACC // L):
                o = v * L
                acc[pl.ds(o, L)] += pv[pl.ds(w * ACC + o, L)]
            return carry

        lax.fori_loop(0, NW, add_worker, 0)

        lane = lax.iota(jnp.int32, L)
        zeros = jnp.zeros((L,), jnp.float32)
        counts_v = zeros
        acc_v = zeros
        conf_v = zeros
        for j in range(N_BINS):
            sel = lane == j
            c = jnp.sum(acc[pl.ds(j * L, L)])
            a = jnp.sum(acc[pl.ds(QSTRIDE + j * L, L)])
            f = jnp.sum(acc[pl.ds(2 * QSTRIDE + j * L, L)])
            counts_v = jnp.where(sel, jnp.broadcast_to(c, (L,)), counts_v)
            acc_v = jnp.where(sel, jnp.broadcast_to(a, (L,)), acc_v)
            conf_v = jnp.where(sel, jnp.broadcast_to(f, (L,)), conf_v)
        ones = jnp.ones((L,), jnp.float32)
        ind = counts_v > jnp.float32(0.0)
        safe = jnp.where(ind, counts_v, ones)
        mean_acc = jnp.where(ind, acc_v / safe, acc_v)
        mean_conf = jnp.where(ind, conf_v / safe, conf_v)
        num = jnp.sum(counts_v * jnp.abs(mean_acc - mean_conf))
        tot = jnp.sum(counts_v)
        outv[...] = jnp.broadcast_to(num, (L,)) / jnp.broadcast_to(tot, (L,))
        pltpu.sync_copy(outv, out_hbm)


@jax.jit
def kernel(yhs, phs, ys):
    bounds = jnp.linspace(0.0, 1.0, N_BINS + 1).astype(jnp.float32)
    parts = _hist_kernel(yhs, phs, ys, bounds)
    ece_vec = _ece_kernel(parts)
    return ece_vec[0]
